# baseline (speedup 1.0000x reference)
import jax
import jax.numpy as jnp
from jax.experimental import pallas as pl
from jax.experimental.pallas import tpu as pltpu

VOCAB = 1000000
EMB = 24
W = 128


def _lookup_kernel(idx_ref, hbm_ref, uv_ref, out_ref, blk_vmem, sem):
    i = idx_ref[()]
    base = pl.multiple_of((i // W) * W, W)
    cp = pltpu.make_async_copy(hbm_ref.at[:, pl.ds(base, W)], blk_vmem, sem)
    cp.start()
    cp.wait()
    prods = jnp.dot(uv_ref[...], blk_vmem[...],
                    preferred_element_type=jnp.float32,
                    precision=jax.lax.Precision.HIGHEST)
    lane = i - base
    mask = jax.lax.broadcasted_iota(jnp.int32, (1, W), 1) == lane
    s = jnp.sum(jnp.where(mask, prods, 0.0), keepdims=True).reshape(1, 1)
    out_ref[...] = jax.nn.sigmoid(s)


def kernel(input, matrix, user_vector):
    idx = jnp.asarray(input, jnp.int32)
    mt = matrix.T
    out = pl.pallas_call(
        _lookup_kernel,
        in_specs=[
            pl.BlockSpec(memory_space=pltpu.SMEM),
            pl.BlockSpec(memory_space=pl.ANY),
            pl.BlockSpec(memory_space=pltpu.VMEM),
        ],
        out_specs=pl.BlockSpec(memory_space=pltpu.VMEM),
        out_shape=jax.ShapeDtypeStruct((1, 1), jnp.float32),
        scratch_shapes=[
            pltpu.VMEM((EMB, W), jnp.float32),
            pltpu.SemaphoreType.DMA,
        ],
    )(idx, mt, user_vector)
    return out.reshape((1,))
